# baseline (speedup 1.0000x reference)
import functools

import jax
import jax.numpy as jnp
from jax import lax
from jax.experimental import pallas as pl
from jax.experimental.pallas import tpu as pltpu
from jax.experimental.pallas import tpu_sc as plsc

N = 256
F = 255
E = 65536
HIDDEN = 2048
OUT_DIM = 32640


_NC = 2
_NS = 16
_NW = _NC * _NS
_EPW = E // _NW
_CSZ = N * N
_ZSL = _CSZ // _NS
_NROW = _EPW // 128


def _sc_counts_body(src_hbm, dst_hbm, out_hbm,
                    src_v, dst_v, idx_v, val_v, zer_v, c_sh):
    cid = lax.axis_index("c")
    sid = lax.axis_index("s")
    wid = sid * _NC + cid
    base = wid * _EPW
    pltpu.sync_copy(src_hbm.at[pl.ds(base, _EPW)], src_v)
    pltpu.sync_copy(dst_hbm.at[pl.ds(base, _EPW)], dst_v)

    zeros = jnp.zeros((16,), jnp.float32)
    ones = jnp.ones((16,), jnp.float32)
    for j in range(_ZSL // 16):
        zer_v[pl.ds(j * 16, 16)] = zeros
    for k in range(8):
        val_v[pl.ds(k * 16, 16)] = ones
    for j in range(_NROW):
        for k in range(8):
            s = pl.ds((j * 8 + k) * 16, 16)
            idx_v[j, pl.ds(k * 16, 16)] = dst_v[s] * 256 + src_v[s]

    pltpu.sync_copy(zer_v, c_sh.at[pl.ds(sid * _ZSL, _ZSL)])
    plsc.subcore_barrier()
    for j in range(_NROW):
        pltpu.sync_copy(val_v, c_sh.at[idx_v.at[j]], add=True)
    plsc.subcore_barrier()
    pltpu.sync_copy(c_sh.at[pl.ds(sid * _ZSL, _ZSL)],
                    out_hbm.at[cid, pl.ds(sid * _ZSL, _ZSL)])


_sc_counts = functools.partial(
    pl.kernel,
    out_type=jax.ShapeDtypeStruct((_NC, _CSZ), jnp.float32),
    mesh=plsc.VectorSubcoreMesh(core_axis_name="c", subcore_axis_name="s"),
    scratch_types=[
        pltpu.VMEM((_EPW,), jnp.int32),
        pltpu.VMEM((_EPW,), jnp.int32),
        pltpu.VMEM((_NROW, 128), jnp.int32),
        pltpu.VMEM((128,), jnp.float32),
        pltpu.VMEM((_ZSL,), jnp.float32),
        pltpu.VMEM_SHARED((_CSZ,), jnp.float32),
    ],
)(_sc_counts_body)


def _gat_layer(h_in, C, mask, W, a_s_row, a_d_col, b_row):
    h = jnp.dot(h_in, W, preferred_element_type=jnp.float32)
    al_s = jax.lax.dot_general(
        a_s_row, h, (((1,), (1,)), ((), ())),
        preferred_element_type=jnp.float32)
    al_d = jnp.dot(h, a_d_col, preferred_element_type=jnp.float32)
    e = al_d + al_s
    e = jnp.where(e >= 0, e, 0.2 * e)
    em = jnp.where(mask, e, -1e30)
    m = jnp.max(em, axis=1, keepdims=True)
    p = jnp.exp(em - m) * C
    denom = jnp.sum(p, axis=1, keepdims=True)
    P = p / (denom + 1e-16)
    return jnp.dot(P, h, preferred_element_type=jnp.float32) + b_row


_R1 = 8
_NK1 = N // _R1


def _gmv1_kernel(c_ref, x_ref,
                 w1_ref, as1_ref, ad1_ref, b1_ref,
                 w2_ref, as2_ref, ad2_ref, b2_ref,
                 w3_ref, as3_ref, ad3_ref, b3_ref,
                 w4_ref, b4_ref, o_ref, g_s):
    k = pl.program_id(0)

    @pl.when(k == 0)
    def _():
        rr = jax.lax.broadcasted_iota(jnp.int32, (N, N), 0)
        cc = jax.lax.broadcasted_iota(jnp.int32, (N, N), 1)
        C = c_ref[0] + c_ref[1] + (rr == cc).astype(jnp.float32)
        mask = C > 0
        h = x_ref[...]
        h = _gat_layer(h, C, mask, w1_ref[...], as1_ref[...], ad1_ref[...],
                       b1_ref[...])
        h = _gat_layer(h, C, mask, w2_ref[...], as2_ref[...], ad2_ref[...],
                       b2_ref[...])
        h = _gat_layer(h, C, mask, w3_ref[...], as3_ref[...], ad3_ref[...],
                       b3_ref[...])
        g_s[...] = jnp.maximum(h, 0.0)

    part = jnp.dot(g_s[pl.ds(k * _R1, 1), :F], w4_ref[0],
                   preferred_element_type=jnp.float32)
    for r in range(1, _R1):
        part += jnp.dot(g_s[pl.ds(k * _R1 + r, 1), :F], w4_ref[r],
                        preferred_element_type=jnp.float32)

    @pl.when(k == 0)
    def _():
        o_ref[...] = part

    @pl.when(k > 0)
    def _():
        o_ref[...] += part

    @pl.when(k == _NK1 - 1)
    def _():
        o_ref[...] = jnp.maximum(o_ref[...] + b4_ref[...], 0.0)


def _run_gat_mv1(counts2, xp, layer_params, W4r, b4):
    flat = []
    for (Wp, a_s, a_d, b) in layer_params:
        flat += [Wp, a_s, a_d, b]
    gat_specs = [pl.BlockSpec(a.shape, lambda k, nd=a.ndim: (0,) * nd)
                 for a in [counts2, xp] + flat]
    return pl.pallas_call(
        _gmv1_kernel,
        grid=(_NK1,),
        in_specs=gat_specs + [
            pl.BlockSpec((_R1, F, HIDDEN), lambda k: (k, 0, 0)),
            pl.BlockSpec((1, HIDDEN), lambda k: (0, 0)),
        ],
        out_specs=pl.BlockSpec((1, HIDDEN), lambda k: (0, 0)),
        out_shape=jax.ShapeDtypeStruct((1, HIDDEN), jnp.float32),
        scratch_shapes=[pltpu.VMEM((N, N), jnp.float32)],
    )(counts2, xp, *flat, W4r, b4)


_K2 = 128
_NK2 = HIDDEN // _K2


def _mv2_kernel(y_ref, w_ref, b_ref, o_ref):
    k = pl.program_id(0)
    part = jnp.dot(y_ref[...], w_ref[...], preferred_element_type=jnp.float32)

    @pl.when(k == 0)
    def _():
        o_ref[...] = part + b_ref[...]

    @pl.when(k > 0)
    def _():
        o_ref[...] += part


def _run_mv2(y1, W5, b5):
    return pl.pallas_call(
        _mv2_kernel,
        grid=(_NK2,),
        in_specs=[
            pl.BlockSpec((1, _K2), lambda k: (0, k)),
            pl.BlockSpec((_K2, OUT_DIM), lambda k: (k, 0)),
            pl.BlockSpec((1, OUT_DIM), lambda k: (0, 0)),
        ],
        out_specs=pl.BlockSpec((1, OUT_DIM), lambda k: (0, 0)),
        out_shape=jax.ShapeDtypeStruct((1, OUT_DIM), jnp.float32),
    )(y1, W5, b5)


def _pad_w(W):
    return jnp.pad(W, ((0, 1), (0, 1)))


def kernel(x, edge_index, W1, a_src1, a_dst1, b1, W2, a_src2, a_dst2, b2,
           W3, a_src3, a_dst3, b3, W4, b4, W5, b5):
    ei = edge_index.astype(jnp.int32)
    counts2 = _sc_counts(ei[0], ei[1]).reshape(_NC, N, N)

    xp = jnp.pad(x, ((0, 0), (0, 1)))
    layer_params = []
    for (W, a_s, a_d, b) in ((W1, a_src1, a_dst1, b1),
                             (W2, a_src2, a_dst2, b2),
                             (W3, a_src3, a_dst3, b3)):
        layer_params.append((
            _pad_w(W),
            jnp.pad(a_s, (0, 1)).reshape(1, N),
            jnp.pad(a_d, (0, 1)).reshape(N, 1),
            jnp.pad(b, (0, 1)).reshape(1, N),
        ))

    W4r = W4.reshape(N, F, HIDDEN)
    y1 = _run_gat_mv1(counts2, xp, layer_params, W4r,
                      b4.reshape(1, HIDDEN))
    y2 = _run_mv2(y1, W5, b5.reshape(1, OUT_DIM))
    return y2.reshape(OUT_DIM)


# speedup vs baseline: 2.4967x; 2.4967x over previous
import functools

import jax
import jax.numpy as jnp
from jax import lax
from jax.experimental import pallas as pl
from jax.experimental.pallas import tpu as pltpu
from jax.experimental.pallas import tpu_sc as plsc

N = 256
F = 255
E = 65536
HIDDEN = 2048
OUT_DIM = 32640


_NC = 2
_NS = 16
_NW = _NC * _NS
_EPW = E // _NW
_CSZ = N * N
_ZSL = _CSZ // _NS
_NROW = _EPW // 128


def _sc_counts_body(src_hbm, dst_hbm, out_hbm,
                    src_v, dst_v, idx_v, val_v, zer_v, c_sh):
    cid = lax.axis_index("c")
    sid = lax.axis_index("s")
    wid = sid * _NC + cid
    base = wid * _EPW
    pltpu.sync_copy(src_hbm.at[pl.ds(base, _EPW)], src_v)
    pltpu.sync_copy(dst_hbm.at[pl.ds(base, _EPW)], dst_v)

    zeros = jnp.zeros((16,), jnp.float32)
    ones = jnp.ones((16,), jnp.float32)
    for j in range(_ZSL // 16):
        zer_v[pl.ds(j * 16, 16)] = zeros
    for k in range(8):
        val_v[pl.ds(k * 16, 16)] = ones
    for j in range(_NROW):
        for k in range(8):
            s = pl.ds((j * 8 + k) * 16, 16)
            idx_v[j, pl.ds(k * 16, 16)] = dst_v[s] * 256 + src_v[s]

    pltpu.sync_copy(zer_v, c_sh.at[pl.ds(sid * _ZSL, _ZSL)])
    plsc.subcore_barrier()
    for j in range(_NROW):
        pltpu.sync_copy(val_v, c_sh.at[idx_v.at[j]], add=True)
    plsc.subcore_barrier()
    pltpu.sync_copy(c_sh.at[pl.ds(sid * _ZSL, _ZSL)],
                    out_hbm.at[cid, pl.ds(sid * _ZSL, _ZSL)])


_sc_counts = functools.partial(
    pl.kernel,
    out_type=jax.ShapeDtypeStruct((_NC, _CSZ), jnp.float32),
    mesh=plsc.VectorSubcoreMesh(core_axis_name="c", subcore_axis_name="s"),
    scratch_types=[
        pltpu.VMEM((_EPW,), jnp.int32),
        pltpu.VMEM((_EPW,), jnp.int32),
        pltpu.VMEM((_NROW, 128), jnp.int32),
        pltpu.VMEM((128,), jnp.float32),
        pltpu.VMEM((_ZSL,), jnp.float32),
        pltpu.VMEM_SHARED((_CSZ,), jnp.float32),
    ],
)(_sc_counts_body)


def _gat_layer(h_in, C, mask, W, a_s_row, a_d_col, b_row):
    h = jnp.dot(h_in, W, preferred_element_type=jnp.float32)
    al_s = jax.lax.dot_general(
        a_s_row, h, (((1,), (1,)), ((), ())),
        preferred_element_type=jnp.float32)
    al_d = jnp.dot(h, a_d_col, preferred_element_type=jnp.float32)
    e = al_d + al_s
    e = jnp.where(e >= 0, e, 0.2 * e)
    em = jnp.where(mask, e, -1e30)
    m = jnp.max(em, axis=1, keepdims=True)
    p = jnp.exp(em - m) * C
    denom = jnp.sum(p, axis=1, keepdims=True)
    P = p / (denom + 1e-16)
    return jnp.dot(P, h, preferred_element_type=jnp.float32) + b_row


def _gat3_kernel(c_ref, x_ref,
                 w1_ref, as1_ref, ad1_ref, b1_ref,
                 w2_ref, as2_ref, ad2_ref, b2_ref,
                 w3_ref, as3_ref, ad3_ref, b3_ref,
                 out_ref):
    rr = jax.lax.broadcasted_iota(jnp.int32, (N, N), 0)
    cc = jax.lax.broadcasted_iota(jnp.int32, (N, N), 1)
    C = c_ref[0] + c_ref[1] + (rr == cc).astype(jnp.float32)
    mask = C > 0
    h = x_ref[...]
    h = _gat_layer(h, C, mask, w1_ref[...], as1_ref[...], ad1_ref[...],
                   b1_ref[...])
    h = _gat_layer(h, C, mask, w2_ref[...], as2_ref[...], ad2_ref[...],
                   b2_ref[...])
    h = _gat_layer(h, C, mask, w3_ref[...], as3_ref[...], ad3_ref[...],
                   b3_ref[...])
    out_ref[...] = jnp.maximum(h[:, :F], 0.0)


def _run_gat3(counts2, xp, layer_params):
    flat = []
    for (Wp, a_s, a_d, b) in layer_params:
        flat += [Wp, a_s, a_d, b]
    return pl.pallas_call(
        _gat3_kernel,
        out_shape=jax.ShapeDtypeStruct((N, F), jnp.float32),
    )(counts2, xp, *flat)


_K1 = 1920
_NK1 = (N * F) // _K1


def _mv1_kernel(y_ref, w_ref, b_ref, o_ref):
    k = pl.program_id(0)
    part = jnp.dot(y_ref[...], w_ref[...], preferred_element_type=jnp.float32)

    @pl.when(k == 0)
    def _():
        o_ref[...] = part

    @pl.when(k > 0)
    def _():
        o_ref[...] += part

    @pl.when(k == _NK1 - 1)
    def _():
        o_ref[...] = jnp.maximum(o_ref[...] + b_ref[...], 0.0)


def _run_mv1(y0, W4, b4):
    return pl.pallas_call(
        _mv1_kernel,
        grid=(_NK1,),
        in_specs=[
            pl.BlockSpec((1, _K1), lambda k: (0, k)),
            pl.BlockSpec((_K1, HIDDEN), lambda k: (k, 0)),
            pl.BlockSpec((1, HIDDEN), lambda k: (0, 0)),
        ],
        out_specs=pl.BlockSpec((1, HIDDEN), lambda k: (0, 0)),
        out_shape=jax.ShapeDtypeStruct((1, HIDDEN), jnp.float32),
    )(y0, W4, b4)


_K2 = 128
_NK2 = HIDDEN // _K2


def _mv2_kernel(y_ref, w_ref, b_ref, o_ref):
    k = pl.program_id(0)
    part = jnp.dot(y_ref[...], w_ref[...], preferred_element_type=jnp.float32)

    @pl.when(k == 0)
    def _():
        o_ref[...] = part + b_ref[...]

    @pl.when(k > 0)
    def _():
        o_ref[...] += part


def _run_mv2(y1, W5, b5):
    return pl.pallas_call(
        _mv2_kernel,
        grid=(_NK2,),
        in_specs=[
            pl.BlockSpec((1, _K2), lambda k: (0, k)),
            pl.BlockSpec((_K2, OUT_DIM), lambda k: (k, 0)),
            pl.BlockSpec((1, OUT_DIM), lambda k: (0, 0)),
        ],
        out_specs=pl.BlockSpec((1, OUT_DIM), lambda k: (0, 0)),
        out_shape=jax.ShapeDtypeStruct((1, OUT_DIM), jnp.float32),
    )(y1, W5, b5)


def _pad_w(W):
    return jnp.pad(W, ((0, 1), (0, 1)))


def kernel(x, edge_index, W1, a_src1, a_dst1, b1, W2, a_src2, a_dst2, b2,
           W3, a_src3, a_dst3, b3, W4, b4, W5, b5):
    ei = edge_index.astype(jnp.int32)
    counts2 = _sc_counts(ei[0], ei[1]).reshape(_NC, N, N)

    xp = jnp.pad(x, ((0, 0), (0, 1)))
    layer_params = []
    for (W, a_s, a_d, b) in ((W1, a_src1, a_dst1, b1),
                             (W2, a_src2, a_dst2, b2),
                             (W3, a_src3, a_dst3, b3)):
        layer_params.append((
            _pad_w(W),
            jnp.pad(a_s, (0, 1)).reshape(1, N),
            jnp.pad(a_d, (0, 1)).reshape(N, 1),
            jnp.pad(b, (0, 1)).reshape(1, N),
        ))

    g3r = _run_gat3(counts2, xp, layer_params)
    y0 = g3r.reshape(1, N * F)
    y1 = _run_mv1(y0, W4, b4.reshape(1, HIDDEN))
    y2 = _run_mv2(y1, W5, b5.reshape(1, OUT_DIM))
    return y2.reshape(OUT_DIM)


# speedup vs baseline: 2.5234x; 1.0107x over previous
import functools

import jax
import jax.numpy as jnp
from jax import lax
from jax.experimental import pallas as pl
from jax.experimental.pallas import tpu as pltpu
from jax.experimental.pallas import tpu_sc as plsc

N = 256
F = 255
E = 65536
HIDDEN = 2048
OUT_DIM = 32640


_NC = 2
_NS = 16
_NW = _NC * _NS
_EPW = E // _NW
_CSZ = N * N
_ZSL = _CSZ // _NS
_NROW = _EPW // 128


def _sc_counts_body(src_hbm, dst_hbm, out_hbm,
                    src_v, dst_v, idx_v, val_v, zer_v, c_sh):
    cid = lax.axis_index("c")
    sid = lax.axis_index("s")
    wid = sid * _NC + cid
    base = wid * _EPW
    pltpu.sync_copy(src_hbm.at[pl.ds(base, _EPW)], src_v)
    pltpu.sync_copy(dst_hbm.at[pl.ds(base, _EPW)], dst_v)

    zeros = jnp.zeros((16,), jnp.float32)
    ones = jnp.ones((16,), jnp.float32)
    for j in range(_ZSL // 16):
        zer_v[pl.ds(j * 16, 16)] = zeros
    for k in range(8):
        val_v[pl.ds(k * 16, 16)] = ones
    for j in range(_NROW):
        for k in range(8):
            s = pl.ds((j * 8 + k) * 16, 16)
            idx_v[j, pl.ds(k * 16, 16)] = dst_v[s] * 256 + src_v[s]

    pltpu.sync_copy(zer_v, c_sh.at[pl.ds(sid * _ZSL, _ZSL)])
    plsc.subcore_barrier()
    for j in range(_NROW):
        pltpu.sync_copy(val_v, c_sh.at[idx_v.at[j]], add=True)
    plsc.subcore_barrier()
    pltpu.sync_copy(c_sh.at[pl.ds(sid * _ZSL, _ZSL)],
                    out_hbm.at[cid, pl.ds(sid * _ZSL, _ZSL)])


_sc_counts = functools.partial(
    pl.kernel,
    out_type=jax.ShapeDtypeStruct((_NC, _CSZ), jnp.float32),
    mesh=plsc.VectorSubcoreMesh(core_axis_name="c", subcore_axis_name="s"),
    scratch_types=[
        pltpu.VMEM((_EPW,), jnp.int32),
        pltpu.VMEM((_EPW,), jnp.int32),
        pltpu.VMEM((_NROW, 128), jnp.int32),
        pltpu.VMEM((128,), jnp.float32),
        pltpu.VMEM((_ZSL,), jnp.float32),
        pltpu.VMEM_SHARED((_CSZ,), jnp.float32),
    ],
)(_sc_counts_body)


def _gat_layer(h_in, C, mask, W, a_s_row, a_d_col, b_row):
    h = jnp.dot(h_in, W, preferred_element_type=jnp.float32)
    al_s = jax.lax.dot_general(
        a_s_row, h, (((1,), (1,)), ((), ())),
        preferred_element_type=jnp.float32)
    al_d = jnp.dot(h, a_d_col, preferred_element_type=jnp.float32)
    e = al_d + al_s
    e = jnp.where(e >= 0, e, 0.2 * e)
    em = jnp.where(mask, e, -1e30)
    m = jnp.max(em, axis=1, keepdims=True)
    p = jnp.exp(em - m) * C
    denom = jnp.sum(p, axis=1, keepdims=True)
    P = p / (denom + 1e-16)
    return jnp.dot(P, h, preferred_element_type=jnp.float32) + b_row


def _gat3_kernel(c_ref, x_ref,
                 w1_ref, as1_ref, ad1_ref, b1_ref,
                 w2_ref, as2_ref, ad2_ref, b2_ref,
                 w3_ref, as3_ref, ad3_ref, b3_ref,
                 out_ref):
    rr = jax.lax.broadcasted_iota(jnp.int32, (N, N), 0)
    cc = jax.lax.broadcasted_iota(jnp.int32, (N, N), 1)
    C = c_ref[0] + c_ref[1] + (rr == cc).astype(jnp.float32)
    mask = C > 0
    h = x_ref[...]
    h = _gat_layer(h, C, mask, w1_ref[...], as1_ref[...], ad1_ref[...],
                   b1_ref[...])
    h = _gat_layer(h, C, mask, w2_ref[...], as2_ref[...], ad2_ref[...],
                   b2_ref[...])
    h = _gat_layer(h, C, mask, w3_ref[...], as3_ref[...], ad3_ref[...],
                   b3_ref[...])
    out_ref[...] = jnp.maximum(h[:, :F], 0.0)


def _run_gat3(counts2, xp, layer_params):
    flat = []
    for (Wp, a_s, a_d, b) in layer_params:
        flat += [Wp, a_s, a_d, b]
    return pl.pallas_call(
        _gat3_kernel,
        out_shape=jax.ShapeDtypeStruct((N, F), jnp.float32),
    )(counts2, xp, *flat)


_K1 = 768
_NK1 = (N * F) // _K1
_N2 = 2176
_NN2 = OUT_DIM // _N2
_NSTEP = _NK1 + _NN2


def _head_kernel(y_ref, w4_ref, b4_ref, w5_ref, b5_ref, o_ref, y1_s):
    k = pl.program_id(0)

    @pl.when(k < _NK1)
    def _():
        part = jnp.dot(y_ref[...], w4_ref[...],
                       preferred_element_type=jnp.float32)

        @pl.when(k == 0)
        def _():
            y1_s[...] = part

        @pl.when(k > 0)
        def _():
            y1_s[...] += part

        @pl.when(k == _NK1 - 1)
        def _():
            y1_s[...] = jnp.maximum(y1_s[...] + b4_ref[...], 0.0)

    @pl.when(k >= _NK1)
    def _():
        o_ref[...] = (
            jnp.dot(y1_s[...], w5_ref[...],
                    preferred_element_type=jnp.float32)
            + b5_ref[...])


def _run_head(y0, W4, b4, W5, b5):
    return pl.pallas_call(
        _head_kernel,
        grid=(_NSTEP,),
        in_specs=[
            pl.BlockSpec((1, _K1), lambda k: (0, jnp.minimum(k, _NK1 - 1))),
            pl.BlockSpec((_K1, HIDDEN),
                         lambda k: (jnp.minimum(k, _NK1 - 1), 0)),
            pl.BlockSpec((1, HIDDEN), lambda k: (0, 0)),
            pl.BlockSpec((HIDDEN, _N2),
                         lambda k: (0, jnp.maximum(k - _NK1, 0))),
            pl.BlockSpec((1, _N2), lambda k: (0, jnp.maximum(k - _NK1, 0))),
        ],
        out_specs=pl.BlockSpec((1, _N2), lambda k: (0, jnp.maximum(k - _NK1, 0))),
        out_shape=jax.ShapeDtypeStruct((1, OUT_DIM), jnp.float32),
        scratch_shapes=[pltpu.VMEM((1, HIDDEN), jnp.float32)],
    )(y0, W4, b4, W5, b5)


def _pad_w(W):
    return jnp.pad(W, ((0, 1), (0, 1)))


def kernel(x, edge_index, W1, a_src1, a_dst1, b1, W2, a_src2, a_dst2, b2,
           W3, a_src3, a_dst3, b3, W4, b4, W5, b5):
    ei = edge_index.astype(jnp.int32)
    counts2 = _sc_counts(ei[0], ei[1]).reshape(_NC, N, N)

    xp = jnp.pad(x, ((0, 0), (0, 1)))
    layer_params = []
    for (W, a_s, a_d, b) in ((W1, a_src1, a_dst1, b1),
                             (W2, a_src2, a_dst2, b2),
                             (W3, a_src3, a_dst3, b3)):
        layer_params.append((
            _pad_w(W),
            jnp.pad(a_s, (0, 1)).reshape(1, N),
            jnp.pad(a_d, (0, 1)).reshape(N, 1),
            jnp.pad(b, (0, 1)).reshape(1, N),
        ))

    g3r = _run_gat3(counts2, xp, layer_params)
    y0 = g3r.reshape(1, N * F)
    y2 = _run_head(y0, W4, b4.reshape(1, HIDDEN),
                   W5, b5.reshape(1, OUT_DIM))
    return y2.reshape(OUT_DIM)


# speedup vs baseline: 2.5244x; 1.0004x over previous
import functools

import jax
import jax.numpy as jnp
from jax import lax
from jax.experimental import pallas as pl
from jax.experimental.pallas import tpu as pltpu
from jax.experimental.pallas import tpu_sc as plsc

N = 256
F = 255
E = 65536
HIDDEN = 2048
OUT_DIM = 32640


_NC = 2
_NS = 16
_NW = _NC * _NS
_EPW = E // _NW
_CSZ = N * N
_ZSL = _CSZ // _NS
_NROW = _EPW // 128


def _sc_counts_body(src_hbm, dst_hbm, out_hbm,
                    src_v, dst_v, idx_v, val_v, zer_v, c_sh):
    cid = lax.axis_index("c")
    sid = lax.axis_index("s")
    wid = sid * _NC + cid
    base = wid * _EPW
    pltpu.sync_copy(src_hbm.at[pl.ds(base, _EPW)], src_v)
    pltpu.sync_copy(dst_hbm.at[pl.ds(base, _EPW)], dst_v)

    zeros = jnp.zeros((16,), jnp.float32)
    ones = jnp.ones((16,), jnp.float32)
    for j in range(_ZSL // 16):
        zer_v[pl.ds(j * 16, 16)] = zeros
    for k in range(8):
        val_v[pl.ds(k * 16, 16)] = ones
    for j in range(_NROW):
        for k in range(8):
            s = pl.ds((j * 8 + k) * 16, 16)
            idx_v[j, pl.ds(k * 16, 16)] = dst_v[s] * 256 + src_v[s]

    pltpu.sync_copy(zer_v, c_sh.at[pl.ds(sid * _ZSL, _ZSL)])
    plsc.subcore_barrier()
    for j in range(_NROW):
        pltpu.sync_copy(val_v, c_sh.at[idx_v.at[j]], add=True)
    plsc.subcore_barrier()
    pltpu.sync_copy(c_sh.at[pl.ds(sid * _ZSL, _ZSL)],
                    out_hbm.at[cid, pl.ds(sid * _ZSL, _ZSL)])


_sc_counts = functools.partial(
    pl.kernel,
    out_type=jax.ShapeDtypeStruct((_NC, _CSZ), jnp.float32),
    mesh=plsc.VectorSubcoreMesh(core_axis_name="c", subcore_axis_name="s"),
    scratch_types=[
        pltpu.VMEM((_EPW,), jnp.int32),
        pltpu.VMEM((_EPW,), jnp.int32),
        pltpu.VMEM((_NROW, 128), jnp.int32),
        pltpu.VMEM((128,), jnp.float32),
        pltpu.VMEM((_ZSL,), jnp.float32),
        pltpu.VMEM_SHARED((_CSZ,), jnp.float32),
    ],
)(_sc_counts_body)


def _gat_layer(h_in, C, mask, W, a_s_row, a_d_col, b_row):
    h = jnp.dot(h_in, W, preferred_element_type=jnp.float32)
    al_s = jax.lax.dot_general(
        a_s_row, h, (((1,), (1,)), ((), ())),
        preferred_element_type=jnp.float32)
    al_d = jnp.dot(h, a_d_col, preferred_element_type=jnp.float32)
    e = al_d + al_s
    e = jnp.where(e >= 0, e, 0.2 * e)
    em = jnp.where(mask, e, -1e30)
    m = jnp.max(em, axis=1, keepdims=True)
    p = jnp.exp(em - m) * C
    denom = jnp.sum(p, axis=1, keepdims=True)
    P = p / (denom + 1e-16)
    return jnp.dot(P, h, preferred_element_type=jnp.float32) + b_row


def _gat3_kernel(c_ref, x_ref,
                 w1_ref, as1_ref, ad1_ref, b1_ref,
                 w2_ref, as2_ref, ad2_ref, b2_ref,
                 w3_ref, as3_ref, ad3_ref, b3_ref,
                 out_ref):
    rr = jax.lax.broadcasted_iota(jnp.int32, (N, N), 0)
    cc = jax.lax.broadcasted_iota(jnp.int32, (N, N), 1)
    C = c_ref[0] + c_ref[1] + (rr == cc).astype(jnp.float32)
    mask = C > 0
    h = x_ref[...]
    h = _gat_layer(h, C, mask, w1_ref[...], as1_ref[...], ad1_ref[...],
                   b1_ref[...])
    h = _gat_layer(h, C, mask, w2_ref[...], as2_ref[...], ad2_ref[...],
                   b2_ref[...])
    h = _gat_layer(h, C, mask, w3_ref[...], as3_ref[...], ad3_ref[...],
                   b3_ref[...])
    out_ref[...] = jnp.maximum(h[:, :F], 0.0)


def _run_gat3(counts2, xp, layer_params):
    flat = []
    for (Wp, a_s, a_d, b) in layer_params:
        flat += [Wp, a_s, a_d, b]
    return pl.pallas_call(
        _gat3_kernel,
        out_shape=jax.ShapeDtypeStruct((N, F), jnp.float32),
    )(counts2, xp, *flat)


_K1 = 1280
_NK1 = (N * F) // _K1
_N2 = 2176
_NN2 = OUT_DIM // _N2
_NSTEP = _NK1 + _NN2


def _head_kernel(y_ref, w4_ref, b4_ref, w5_ref, b5_ref, o_ref, y1_s):
    k = pl.program_id(0)

    @pl.when(k < _NK1)
    def _():
        part = jnp.dot(y_ref[...], w4_ref[...],
                       preferred_element_type=jnp.float32)

        @pl.when(k == 0)
        def _():
            y1_s[...] = part

        @pl.when(k > 0)
        def _():
            y1_s[...] += part

        @pl.when(k == _NK1 - 1)
        def _():
            y1_s[...] = jnp.maximum(y1_s[...] + b4_ref[...], 0.0)

    @pl.when(k >= _NK1)
    def _():
        o_ref[...] = (
            jnp.dot(y1_s[...], w5_ref[...],
                    preferred_element_type=jnp.float32)
            + b5_ref[...])


def _run_head(y0, W4, b4, W5, b5):
    return pl.pallas_call(
        _head_kernel,
        grid=(_NSTEP,),
        in_specs=[
            pl.BlockSpec((1, _K1), lambda k: (0, jnp.minimum(k, _NK1 - 1))),
            pl.BlockSpec((_K1, HIDDEN),
                         lambda k: (jnp.minimum(k, _NK1 - 1), 0)),
            pl.BlockSpec((1, HIDDEN), lambda k: (0, 0)),
            pl.BlockSpec((HIDDEN, _N2),
                         lambda k: (0, jnp.maximum(k - _NK1, 0))),
            pl.BlockSpec((1, _N2), lambda k: (0, jnp.maximum(k - _NK1, 0))),
        ],
        out_specs=pl.BlockSpec((1, _N2), lambda k: (0, jnp.maximum(k - _NK1, 0))),
        out_shape=jax.ShapeDtypeStruct((1, OUT_DIM), jnp.float32),
        scratch_shapes=[pltpu.VMEM((1, HIDDEN), jnp.float32)],
    )(y0, W4, b4, W5, b5)


def _pad_w(W):
    return jnp.pad(W, ((0, 1), (0, 1)))


def kernel(x, edge_index, W1, a_src1, a_dst1, b1, W2, a_src2, a_dst2, b2,
           W3, a_src3, a_dst3, b3, W4, b4, W5, b5):
    ei = edge_index.astype(jnp.int32)
    counts2 = _sc_counts(ei[0], ei[1]).reshape(_NC, N, N)

    xp = jnp.pad(x, ((0, 0), (0, 1)))
    layer_params = []
    for (W, a_s, a_d, b) in ((W1, a_src1, a_dst1, b1),
                             (W2, a_src2, a_dst2, b2),
                             (W3, a_src3, a_dst3, b3)):
        layer_params.append((
            _pad_w(W),
            jnp.pad(a_s, (0, 1)).reshape(1, N),
            jnp.pad(a_d, (0, 1)).reshape(N, 1),
            jnp.pad(b, (0, 1)).reshape(1, N),
        ))

    g3r = _run_gat3(counts2, xp, layer_params)
    y0 = g3r.reshape(1, N * F)
    y2 = _run_head(y0, W4, b4.reshape(1, HIDDEN),
                   W5, b5.reshape(1, OUT_DIM))
    return y2.reshape(OUT_DIM)


# speedup vs baseline: 2.5256x; 1.0005x over previous
import functools

import jax
import jax.numpy as jnp
from jax import lax
from jax.experimental import pallas as pl
from jax.experimental.pallas import tpu as pltpu
from jax.experimental.pallas import tpu_sc as plsc

N = 256
F = 255
E = 65536
HIDDEN = 2048
OUT_DIM = 32640


_NC = 2
_NS = 16
_NW = _NC * _NS
_EPW = E // _NW
_CSZ = N * N
_ZSL = _CSZ // _NS
_NROW = _EPW // 128


def _sc_counts_body(src_hbm, dst_hbm, out_hbm,
                    src_v, dst_v, idx_v, val_v, zer_v, c_sh, sem):
    cid = lax.axis_index("c")
    sid = lax.axis_index("s")
    wid = sid * _NC + cid
    base = wid * _EPW
    pltpu.sync_copy(src_hbm.at[pl.ds(base, _EPW)], src_v)
    pltpu.sync_copy(dst_hbm.at[pl.ds(base, _EPW)], dst_v)

    zeros = jnp.zeros((16,), jnp.float32)
    ones = jnp.ones((16,), jnp.float32)
    for j in range(_ZSL // 16):
        zer_v[pl.ds(j * 16, 16)] = zeros
    for k in range(8):
        val_v[pl.ds(k * 16, 16)] = ones
    for j in range(_NROW):
        for k in range(8):
            s = pl.ds((j * 8 + k) * 16, 16)
            idx_v[j, pl.ds(k * 16, 16)] = dst_v[s] * 256 + src_v[s]

    pltpu.sync_copy(zer_v, c_sh.at[pl.ds(sid * _ZSL, _ZSL)])
    plsc.subcore_barrier()
    cps = [pltpu.async_copy(val_v, c_sh.at[idx_v.at[j]], sem, add=True)
           for j in range(_NROW)]
    for cp in cps:
        cp.wait()
    plsc.subcore_barrier()
    pltpu.sync_copy(c_sh.at[pl.ds(sid * _ZSL, _ZSL)],
                    out_hbm.at[cid, pl.ds(sid * _ZSL, _ZSL)])


_sc_counts = functools.partial(
    pl.kernel,
    out_type=jax.ShapeDtypeStruct((_NC, _CSZ), jnp.float32),
    mesh=plsc.VectorSubcoreMesh(core_axis_name="c", subcore_axis_name="s"),
    scratch_types=[
        pltpu.VMEM((_EPW,), jnp.int32),
        pltpu.VMEM((_EPW,), jnp.int32),
        pltpu.VMEM((_NROW, 128), jnp.int32),
        pltpu.VMEM((128,), jnp.float32),
        pltpu.VMEM((_ZSL,), jnp.float32),
        pltpu.VMEM_SHARED((_CSZ,), jnp.float32),
        pltpu.SemaphoreType.DMA,
    ],
)(_sc_counts_body)


def _gat_layer(h_in, C, mask, W, a_s_row, a_d_col, b_row):
    h = jnp.dot(h_in, W, preferred_element_type=jnp.float32)
    al_s = jax.lax.dot_general(
        a_s_row, h, (((1,), (1,)), ((), ())),
        preferred_element_type=jnp.float32)
    al_d = jnp.dot(h, a_d_col, preferred_element_type=jnp.float32)
    e = al_d + al_s
    e = jnp.where(e >= 0, e, 0.2 * e)
    em = jnp.where(mask, e, -1e30)
    m = jnp.max(em, axis=1, keepdims=True)
    p = jnp.exp(em - m) * C
    denom = jnp.sum(p, axis=1, keepdims=True)
    P = p / (denom + 1e-16)
    return jnp.dot(P, h, preferred_element_type=jnp.float32) + b_row


def _gat3_kernel(c_ref, x_ref,
                 w1_ref, as1_ref, ad1_ref, b1_ref,
                 w2_ref, as2_ref, ad2_ref, b2_ref,
                 w3_ref, as3_ref, ad3_ref, b3_ref,
                 out_ref):
    rr = jax.lax.broadcasted_iota(jnp.int32, (N, N), 0)
    cc = jax.lax.broadcasted_iota(jnp.int32, (N, N), 1)
    C = c_ref[0] + c_ref[1] + (rr == cc).astype(jnp.float32)
    mask = C > 0
    h = x_ref[...]
    h = _gat_layer(h, C, mask, w1_ref[...], as1_ref[...], ad1_ref[...],
                   b1_ref[...])
    h = _gat_layer(h, C, mask, w2_ref[...], as2_ref[...], ad2_ref[...],
                   b2_ref[...])
    h = _gat_layer(h, C, mask, w3_ref[...], as3_ref[...], ad3_ref[...],
                   b3_ref[...])
    out_ref[...] = jnp.maximum(h[:, :F], 0.0)


def _run_gat3(counts2, xp, layer_params):
    flat = []
    for (Wp, a_s, a_d, b) in layer_params:
        flat += [Wp, a_s, a_d, b]
    return pl.pallas_call(
        _gat3_kernel,
        out_shape=jax.ShapeDtypeStruct((N, F), jnp.float32),
    )(counts2, xp, *flat)


_K1 = 1280
_NK1 = (N * F) // _K1
_N2 = 2176
_NN2 = OUT_DIM // _N2
_NSTEP = _NK1 + _NN2


def _head_kernel(y_ref, w4_ref, b4_ref, w5_ref, b5_ref, o_ref, y1_s):
    k = pl.program_id(0)

    @pl.when(k < _NK1)
    def _():
        part = jnp.dot(y_ref[...], w4_ref[...],
                       preferred_element_type=jnp.float32)

        @pl.when(k == 0)
        def _():
            y1_s[...] = part

        @pl.when(k > 0)
        def _():
            y1_s[...] += part

        @pl.when(k == _NK1 - 1)
        def _():
            y1_s[...] = jnp.maximum(y1_s[...] + b4_ref[...], 0.0)

    @pl.when(k >= _NK1)
    def _():
        o_ref[...] = (
            jnp.dot(y1_s[...], w5_ref[...],
                    preferred_element_type=jnp.float32)
            + b5_ref[...])


def _run_head(y0, W4, b4, W5, b5):
    return pl.pallas_call(
        _head_kernel,
        grid=(_NSTEP,),
        in_specs=[
            pl.BlockSpec((1, _K1), lambda k: (0, jnp.minimum(k, _NK1 - 1))),
            pl.BlockSpec((_K1, HIDDEN),
                         lambda k: (jnp.minimum(k, _NK1 - 1), 0)),
            pl.BlockSpec((1, HIDDEN), lambda k: (0, 0)),
            pl.BlockSpec((HIDDEN, _N2),
                         lambda k: (0, jnp.maximum(k - _NK1, 0))),
            pl.BlockSpec((1, _N2), lambda k: (0, jnp.maximum(k - _NK1, 0))),
        ],
        out_specs=pl.BlockSpec((1, _N2), lambda k: (0, jnp.maximum(k - _NK1, 0))),
        out_shape=jax.ShapeDtypeStruct((1, OUT_DIM), jnp.float32),
        scratch_shapes=[pltpu.VMEM((1, HIDDEN), jnp.float32)],
    )(y0, W4, b4, W5, b5)


def _pad_w(W):
    return jnp.pad(W, ((0, 1), (0, 1)))


def kernel(x, edge_index, W1, a_src1, a_dst1, b1, W2, a_src2, a_dst2, b2,
           W3, a_src3, a_dst3, b3, W4, b4, W5, b5):
    ei = edge_index.astype(jnp.int32)
    counts2 = _sc_counts(ei[0], ei[1]).reshape(_NC, N, N)

    xp = jnp.pad(x, ((0, 0), (0, 1)))
    layer_params = []
    for (W, a_s, a_d, b) in ((W1, a_src1, a_dst1, b1),
                             (W2, a_src2, a_dst2, b2),
                             (W3, a_src3, a_dst3, b3)):
        layer_params.append((
            _pad_w(W),
            jnp.pad(a_s, (0, 1)).reshape(1, N),
            jnp.pad(a_d, (0, 1)).reshape(N, 1),
            jnp.pad(b, (0, 1)).reshape(1, N),
        ))

    g3r = _run_gat3(counts2, xp, layer_params)
    y0 = g3r.reshape(1, N * F)
    y2 = _run_head(y0, W4, b4.reshape(1, HIDDEN),
                   W5, b5.reshape(1, OUT_DIM))
    return y2.reshape(OUT_DIM)
